# baseline (speedup 1.0000x reference)
import functools

import jax
import jax.numpy as jnp
from jax import lax
from jax.experimental import pallas as pl
from jax.experimental.pallas import tpu as pltpu
from jax.experimental.pallas import tpu_sc as plsc

N = 50000
D = 128
FE = 9
NF = 128
G = 32
EPS = 1e-5

NW = 32
NP = 50176
PW = NP // NW
C = 112
NCHUNK = PW // C
NIT = NCHUNK * FE
NBUF = 7

STATS_BN = 2000
MM_BN = 512


def _stats_body(x_ref, sum_ref, sq_ref):
    i = pl.program_id(0)
    x = x_ref[...]
    s = jnp.sum(x, axis=0, keepdims=True)
    q = jnp.sum(x * x, axis=0, keepdims=True)

    @pl.when(i == 0)
    def _():
        sum_ref[...] = s
        sq_ref[...] = q

    @pl.when(i != 0)
    def _():
        sum_ref[...] += s
        sq_ref[...] += q


def _mm_body(r3_ref, scale_ref, shift_ref, w_ref, b_ref, o_ref):
    o_ref[...] = jnp.zeros((MM_BN, NF), jnp.float32) + b_ref[...]
    for k in range(FE):
        x = r3_ref[k]
        xn = jnp.maximum(x * scale_ref[...] + shift_ref[...], 0.0)
        xb = xn.astype(jnp.bfloat16)
        o_ref[...] += lax.dot_general(xb, w_ref[k], (((1,), (0,)), ((), ())),
                                      preferred_element_type=jnp.float32)


def _sc_body(tbl_hbm, idx_hbm, rows_hbm, idx_all, bufs, sgs, sws):
    wid = lax.axis_index("s") * 2 + lax.axis_index("c")
    base = wid * PW
    pltpu.sync_copy(idx_hbm.at[pl.ds(wid * (FE * PW), FE * PW)], idx_all)

    def slots(it):
        k = it % FE
        ci = it // FE
        return k * PW + ci * C, k * NP + base + ci * C

    @pl.loop(0, NIT, step=NBUF)
    def _(it0):
        for p in range(NBUF):
            it = it0 + p

            @pl.when(it >= NBUF)
            def _():
                pltpu.make_async_copy(
                    bufs[p], rows_hbm.at[pl.ds(0, C)], sws[p]).wait()

            voff, _ = slots(it)
            pltpu.async_copy(
                tbl_hbm.at[idx_all.at[pl.ds(voff, C)]], bufs[p], sgs[p])
        for p in range(NBUF):
            it = it0 + p
            voff, hoff = slots(it)
            pltpu.make_async_copy(
                tbl_hbm.at[idx_all.at[pl.ds(voff, C)]], bufs[p], sgs[p]).wait()
            pltpu.async_copy(bufs[p], rows_hbm.at[pl.ds(hoff, C)], sws[p])

    for p in range(NBUF):
        pltpu.make_async_copy(bufs[p], rows_hbm.at[pl.ds(0, C)], sws[p]).wait()


def kernel(lv, neighbor_idx, gamma, beta, W, b):
    f32 = jnp.float32

    idx = neighbor_idx.astype(jnp.int32)
    idx_wm = jnp.pad(idx, ((0, NP - N), (0, 0))).T
    idx_wm = idx_wm.reshape(FE, NW, PW).transpose(1, 0, 2).reshape(-1)

    mesh = plsc.VectorSubcoreMesh(core_axis_name="c", subcore_axis_name="s")
    sc_gather = pl.kernel(
        _sc_body,
        out_type=jax.ShapeDtypeStruct((FE * NP, D), f32),
        mesh=mesh,
        scratch_types=[
            pltpu.VMEM((FE * PW,), jnp.int32),
            [pltpu.VMEM((C, D), f32)] * NBUF,
            [pltpu.SemaphoreType.DMA] * NBUF,
            [pltpu.SemaphoreType.DMA] * NBUF,
        ],
    )
    rows3 = sc_gather(lv, idx_wm).reshape(FE, NP, D)

    sums, sqs = pl.pallas_call(
        _stats_body,
        grid=(N // STATS_BN,),
        in_specs=[pl.BlockSpec((STATS_BN, D), lambda i: (i, 0))],
        out_specs=[pl.BlockSpec((1, D), lambda i: (0, 0))] * 2,
        out_shape=[jax.ShapeDtypeStruct((1, D), f32)] * 2,
    )(lv)

    cs = sums.reshape(G, D // G)
    cq = sqs.reshape(G, D // G)
    cnt = f32(N * (D // G))
    mean = cs.sum(1) / cnt
    var = cq.sum(1) / cnt - mean * mean
    rstd = lax.rsqrt(var + EPS)
    g2 = gamma.reshape(G, D // G)
    b2 = beta.reshape(G, D // G)
    scale = (g2 * rstd[:, None]).reshape(1, D)
    shift = (b2 - g2 * (mean * rstd)[:, None]).reshape(1, D)

    w3 = W.reshape(FE, D, NF).astype(jnp.bfloat16)
    out = pl.pallas_call(
        _mm_body,
        grid=(NP // MM_BN,),
        in_specs=[
            pl.BlockSpec((FE, MM_BN, D), lambda i: (0, i, 0)),
            pl.BlockSpec((1, D), lambda i: (0, 0)),
            pl.BlockSpec((1, D), lambda i: (0, 0)),
            pl.BlockSpec((FE, D, NF), lambda i: (0, 0, 0)),
            pl.BlockSpec((1, NF), lambda i: (0, 0)),
        ],
        out_specs=pl.BlockSpec((MM_BN, NF), lambda i: (i, 0)),
        out_shape=jax.ShapeDtypeStruct((NP, NF), f32),
    )(rows3, scale, shift, w3, b.reshape(1, NF))
    return out[:N]


# speedup vs baseline: 1.0064x; 1.0064x over previous
import functools

import jax
import jax.numpy as jnp
from jax import lax
from jax.experimental import pallas as pl
from jax.experimental.pallas import tpu as pltpu
from jax.experimental.pallas import tpu_sc as plsc

N = 50000
D = 128
FE = 9
NF = 128
G = 32
EPS = 1e-5

NW = 32
NP = 50176
S = 4
NPS = NP // S
PWS = NPS // NW
C = 56
NCHUNK = PWS // C
NIT = NCHUNK * FE
NBUF = 9

STATS_BN = 5000
MM_BN = 784


def _stats_body(x_ref, sum_ref, sq_ref):
    i = pl.program_id(0)
    x = x_ref[...]
    s = jnp.sum(x, axis=0, keepdims=True)
    q = jnp.sum(x * x, axis=0, keepdims=True)

    @pl.when(i == 0)
    def _():
        sum_ref[...] = s
        sq_ref[...] = q

    @pl.when(i != 0)
    def _():
        sum_ref[...] += s
        sq_ref[...] += q


def _mm_body(r3_ref, scale_ref, shift_ref, w_ref, b_ref, o_ref):
    o_ref[...] = jnp.zeros((MM_BN, NF), jnp.float32) + b_ref[...]
    for k in range(FE):
        x = r3_ref[k]
        xn = jnp.maximum(x * scale_ref[...] + shift_ref[...], 0.0)
        xb = xn.astype(jnp.bfloat16)
        o_ref[...] += lax.dot_general(xb, w_ref[k], (((1,), (0,)), ((), ())),
                                      preferred_element_type=jnp.float32)


def _sc_body(tbl_hbm, idx_hbm, rows_hbm, idx_all, bufs, sgs, sws):
    wid = lax.axis_index("s") * 2 + lax.axis_index("c")
    base = wid * PWS
    pltpu.sync_copy(idx_hbm.at[pl.ds(wid * (FE * PWS), FE * PWS)], idx_all)

    def slots(it):
        k = it % FE
        ci = it // FE
        return k * PWS + ci * C, k * NPS + base + ci * C

    @pl.loop(0, NIT, step=NBUF)
    def _(it0):
        for p in range(NBUF):
            it = it0 + p

            @pl.when(it >= NBUF)
            def _():
                pltpu.make_async_copy(
                    bufs[p], rows_hbm.at[pl.ds(0, C)], sws[p]).wait()

            voff, _ = slots(it)
            pltpu.async_copy(
                tbl_hbm.at[idx_all.at[pl.ds(voff, C)]], bufs[p], sgs[p])
        for p in range(NBUF):
            it = it0 + p
            voff, hoff = slots(it)
            pltpu.make_async_copy(
                tbl_hbm.at[idx_all.at[pl.ds(voff, C)]], bufs[p], sgs[p]).wait()
            pltpu.async_copy(bufs[p], rows_hbm.at[pl.ds(hoff, C)], sws[p])

    for p in range(NBUF):
        pltpu.make_async_copy(bufs[p], rows_hbm.at[pl.ds(0, C)], sws[p]).wait()


def kernel(lv, neighbor_idx, gamma, beta, W, b):
    f32 = jnp.float32

    idx = neighbor_idx.astype(jnp.int32)
    idxp = jnp.pad(idx, ((0, NP - N), (0, 0)))
    idx_sm = idxp.reshape(S, NW, PWS, FE).transpose(0, 1, 3, 2).reshape(S, -1)

    mesh = plsc.VectorSubcoreMesh(core_axis_name="c", subcore_axis_name="s")
    sc_gather = pl.kernel(
        _sc_body,
        out_type=jax.ShapeDtypeStruct((FE * NPS, D), f32),
        mesh=mesh,
        scratch_types=[
            pltpu.VMEM((FE * PWS,), jnp.int32),
            [pltpu.VMEM((C, D), f32)] * NBUF,
            [pltpu.SemaphoreType.DMA] * NBUF,
            [pltpu.SemaphoreType.DMA] * NBUF,
        ],
    )
    rows3s = [sc_gather(lv, idx_sm[s]).reshape(FE, NPS, D) for s in range(S)]

    sums, sqs = pl.pallas_call(
        _stats_body,
        grid=(N // STATS_BN,),
        in_specs=[pl.BlockSpec((STATS_BN, D), lambda i: (i, 0))],
        out_specs=[pl.BlockSpec((1, D), lambda i: (0, 0))] * 2,
        out_shape=[jax.ShapeDtypeStruct((1, D), f32)] * 2,
    )(lv)

    cs = sums.reshape(G, D // G)
    cq = sqs.reshape(G, D // G)
    cnt = f32(N * (D // G))
    mean = cs.sum(1) / cnt
    var = cq.sum(1) / cnt - mean * mean
    rstd = lax.rsqrt(var + EPS)
    g2 = gamma.reshape(G, D // G)
    b2 = beta.reshape(G, D // G)
    scale = (g2 * rstd[:, None]).reshape(1, D)
    shift = (b2 - g2 * (mean * rstd)[:, None]).reshape(1, D)

    w3 = W.reshape(FE, D, NF).astype(jnp.bfloat16)
    b2d = b.reshape(1, NF)
    outs = []
    for s in range(S):
        nrows = min(NPS, N - s * NPS)
        out_s = pl.pallas_call(
            _mm_body,
            grid=(pl.cdiv(nrows, MM_BN),),
            in_specs=[
                pl.BlockSpec((FE, MM_BN, D), lambda i: (0, i, 0)),
                pl.BlockSpec((1, D), lambda i: (0, 0)),
                pl.BlockSpec((1, D), lambda i: (0, 0)),
                pl.BlockSpec((FE, D, NF), lambda i: (0, 0, 0)),
                pl.BlockSpec((1, NF), lambda i: (0, 0)),
            ],
            out_specs=pl.BlockSpec((MM_BN, NF), lambda i: (i, 0)),
            out_shape=jax.ShapeDtypeStruct((nrows, NF), f32),
        )(rows3s[s], scale, shift, w3, b2d)
        outs.append(out_s)
    return jnp.concatenate(outs, axis=0)


# speedup vs baseline: 1.0852x; 1.0782x over previous
import functools

import jax
import jax.numpy as jnp
from jax import lax
from jax.experimental import pallas as pl
from jax.experimental.pallas import tpu as pltpu
from jax.experimental.pallas import tpu_sc as plsc

N = 50000
D = 128
FE = 9
NF = 128
G = 32
EPS = 1e-5

NW = 32
NP = 50176
S = 4
NPS = NP // S
PWS = NPS // NW
C = 56
NCHUNK = PWS // C
NIT = NCHUNK * FE
NBUF = 9

STATS_BN = 5000
MM_BN = 784


def _stats_body(x_ref, sum_ref, sq_ref):
    i = pl.program_id(0)
    x = x_ref[...]
    s = jnp.sum(x, axis=0, keepdims=True)
    q = jnp.sum(x * x, axis=0, keepdims=True)

    @pl.when(i == 0)
    def _():
        sum_ref[...] = s
        sq_ref[...] = q

    @pl.when(i != 0)
    def _():
        sum_ref[...] += s
        sq_ref[...] += q


def _mm_body(r3_ref, scale_ref, shift_ref, w_ref, b_ref, o_ref):
    o_ref[...] = jnp.zeros((MM_BN, NF), jnp.float32) + b_ref[...]
    for k in range(FE):
        x = r3_ref[k]
        xn = jnp.maximum(x * scale_ref[...] + shift_ref[...], 0.0)
        xb = xn.astype(jnp.bfloat16)
        o_ref[...] += lax.dot_general(xb, w_ref[k], (((1,), (0,)), ((), ())),
                                      preferred_element_type=jnp.float32)


def _sc_body(tbl_hbm, idx_hbm, rows_hbm, idx_all, bufs, sgs, sws):
    wid = lax.axis_index("s") * 2 + lax.axis_index("c")
    base = wid * PWS
    pltpu.sync_copy(idx_hbm.at[pl.ds(wid * (FE * PWS), FE * PWS)], idx_all)

    def slots(it):
        k = it % FE
        ci = it // FE
        return k * PWS + ci * C, k * NPS + base + ci * C

    @pl.loop(0, NIT, step=NBUF)
    def _(it0):
        for p in range(NBUF):
            it = it0 + p

            @pl.when(it >= NBUF)
            def _():
                pltpu.make_async_copy(
                    bufs[p], rows_hbm.at[pl.ds(0, C)], sws[p]).wait()

            voff, _ = slots(it)
            pltpu.async_copy(
                tbl_hbm.at[idx_all.at[pl.ds(voff, C)]], bufs[p], sgs[p])
        for p in range(NBUF):
            it = it0 + p
            voff, hoff = slots(it)
            pltpu.make_async_copy(
                tbl_hbm.at[idx_all.at[pl.ds(voff, C)]], bufs[p], sgs[p]).wait()
            pltpu.async_copy(bufs[p], rows_hbm.at[pl.ds(hoff, C)], sws[p])

    for p in range(NBUF):
        pltpu.make_async_copy(bufs[p], rows_hbm.at[pl.ds(0, C)], sws[p]).wait()


def kernel(lv, neighbor_idx, gamma, beta, W, b):
    f32 = jnp.float32

    idx = neighbor_idx.astype(jnp.int32)
    idxp = jnp.pad(idx, ((0, NP - N), (0, 0)))
    idx_sm = idxp.reshape(S, NW, PWS, FE).transpose(0, 1, 3, 2).reshape(S, -1)

    mesh = plsc.VectorSubcoreMesh(core_axis_name="c", subcore_axis_name="s")
    sc_gather = pl.kernel(
        _sc_body,
        out_type=jax.ShapeDtypeStruct((FE * NPS, D), f32),
        mesh=mesh,
        scratch_types=[
            pltpu.VMEM((FE * PWS,), jnp.int32),
            [pltpu.VMEM((C, D), f32)] * NBUF,
            [pltpu.SemaphoreType.DMA] * NBUF,
            [pltpu.SemaphoreType.DMA] * NBUF,
        ],
    )
    rows3s = [sc_gather(lv, idx_sm[s]).reshape(FE, NPS, D) for s in range(S)]

    sums, sqs = pl.pallas_call(
        _stats_body,
        grid=(N // STATS_BN,),
        in_specs=[pl.BlockSpec((STATS_BN, D), lambda i: (i, 0))],
        out_specs=[pl.BlockSpec((1, D), lambda i: (0, 0))] * 2,
        out_shape=[jax.ShapeDtypeStruct((1, D), f32)] * 2,
    )(lv)

    cs = sums.reshape(G, D // G)
    cq = sqs.reshape(G, D // G)
    cnt = f32(N * (D // G))
    mean = cs.sum(1) / cnt
    var = cq.sum(1) / cnt - mean * mean
    rstd = lax.rsqrt(var + EPS)
    g2 = gamma.reshape(G, D // G)
    b2 = beta.reshape(G, D // G)
    scale = (g2 * rstd[:, None]).reshape(1, D)
    shift = (b2 - g2 * (mean * rstd)[:, None]).reshape(1, D)

    w3 = W.reshape(FE, D, NF).astype(jnp.bfloat16)
    b2d = b.reshape(1, NF)
    out = None
    for s in range(S):
        nrows = min(NPS, N - s * NPS)
        nblk = pl.cdiv(nrows, MM_BN)
        base_blk = s * (NPS // MM_BN)
        body = _mm_body if out is None else (
            lambda r3, sc, sh, w, bb, prev, o: _mm_body(r3, sc, sh, w, bb, o))
        in_specs = [
            pl.BlockSpec((FE, MM_BN, D), lambda i: (0, i, 0)),
            pl.BlockSpec((1, D), lambda i: (0, 0)),
            pl.BlockSpec((1, D), lambda i: (0, 0)),
            pl.BlockSpec((FE, D, NF), lambda i: (0, 0, 0)),
            pl.BlockSpec((1, NF), lambda i: (0, 0)),
        ]
        args = [rows3s[s], scale, shift, w3, b2d]
        aliases = {}
        if out is not None:
            in_specs.append(pl.BlockSpec(memory_space=pltpu.MemorySpace.HBM))
            args.append(out)
            aliases = {5: 0}
        out = pl.pallas_call(
            body,
            grid=(nblk,),
            in_specs=in_specs,
            out_specs=pl.BlockSpec(
                (MM_BN, NF),
                functools.partial(lambda i, bb: (bb + i, 0), bb=base_blk)),
            out_shape=jax.ShapeDtypeStruct((N, NF), f32),
            input_output_aliases=aliases,
        )(*args)
    return out
